# baseline (speedup 1.0000x reference)
import functools

import jax
import jax.numpy as jnp
from jax import lax
from jax.experimental import pallas as pl
from jax.experimental.pallas import tpu as pltpu
from jax.experimental.pallas import tpu_sc as plsc

_N = 10000
_D = 128
_OD = 2
_NP = 10240
_NC, _NS = 2, 16
_NW = _NC * _NS
_CH = 128
_E = 320000
_EW = _E // _NW
_KF = _EW // _CH
_TL = _EW - _KF * _CH
_OW = 16
_RB = 512
_RPS = _NP // _NS


def _mesh():
    return plsc.VectorSubcoreMesh(core_axis_name="c", subcore_axis_name="s",
                                  num_cores=_NC, num_subcores=_NS)


def _deg_sc(dst, z1, one):
    @functools.partial(
        pl.kernel,
        out_type=jax.ShapeDtypeStruct((_NC, _NP), jnp.float32),
        mesh=_mesh(),
        scratch_types=[
            pltpu.VMEM((_CH,), jnp.float32),
            pltpu.VMEM((_CH,), jnp.int32),
            pltpu.VMEM((_CH,), jnp.int32),
            pltpu.VMEM((_TL,), jnp.int32),
            pltpu.SemaphoreType.DMA,
            pltpu.SemaphoreType.DMA,
            pltpu.VMEM_SHARED((_NP,), jnp.float32),
        ],
    )
    def k(dst_hbm, z_hbm, one_hbm, out_hbm,
          ones_v, id0_v, id1_v, idt_v, dsem0, dsem1, deg_sh):
        c = lax.axis_index("c")
        s = lax.axis_index("s")
        w = c * _NS + s
        base = w * _EW
        pltpu.sync_copy(one_hbm, ones_v)
        pltpu.sync_copy(dst_hbm.at[pl.ds(base + _KF * _CH, _TL)], idt_v)
        sl = pl.ds(s * _RPS, _RPS)
        pltpu.sync_copy(z_hbm.at[sl], deg_sh.at[sl])
        plsc.subcore_barrier()

        pltpu.async_copy(dst_hbm.at[pl.ds(base, _CH)], id0_v, dsem0)
        pltpu.async_copy(dst_hbm.at[pl.ds(base + _CH, _CH)], id1_v, dsem1)

        def half(j, id_v, dsem):
            pltpu.make_async_copy(
                dst_hbm.at[pl.ds(base + j * _CH, _CH)], id_v, dsem).wait()
            pltpu.sync_copy(ones_v, deg_sh.at[id_v], add=True)

            @pl.when(j + 2 < _KF)
            def _():
                pltpu.async_copy(
                    dst_hbm.at[pl.ds(base + (j + 2) * _CH, _CH)], id_v, dsem)

        def step(jj, carry):
            half(2 * jj, id0_v, dsem0)
            half(2 * jj + 1, id1_v, dsem1)
            return carry

        lax.fori_loop(0, _KF // 2, step, 0)
        pltpu.sync_copy(ones_v.at[pl.ds(0, _TL)], deg_sh.at[idt_v], add=True)
        plsc.subcore_barrier()
        pltpu.sync_copy(deg_sh.at[sl], out_hbm.at[c, sl])

    return k(dst, z1, one)


def _scatter_sc(src, dst, feat, zeros, width):
    @functools.partial(
        pl.kernel,
        out_type=jax.ShapeDtypeStruct((_NC, _NP, width), jnp.float32),
        mesh=_mesh(),
        compiler_params=pltpu.CompilerParams(
            use_tc_tiling_on_sc=(width == _D)),
        scratch_types=[
            pltpu.VMEM((_KF * _CH,), jnp.int32),
            pltpu.VMEM((_TL,), jnp.int32),
            pltpu.VMEM((_TL,), jnp.int32),
            pltpu.VMEM((_CH,), jnp.int32),
            pltpu.VMEM((_CH,), jnp.int32),
            pltpu.VMEM((_CH, width), jnp.float32),
            pltpu.VMEM((_CH, width), jnp.float32),
            pltpu.VMEM((_TL, width), jnp.float32),
            pltpu.SemaphoreType.DMA,
            pltpu.SemaphoreType.DMA,
            pltpu.SemaphoreType.DMA,
            pltpu.SemaphoreType.DMA,
            pltpu.VMEM_SHARED((_NP, width), jnp.float32),
        ],
    )
    def k(src_hbm, dst_hbm, feat_hbm, z_hbm, out_hbm,
          isrc_v, ist_v, idt_v, id0_v, id1_v, rows0_v, rows1_v, rowst_v,
          gsem0, gsem1, dsem0, dsem1, acc_sh):
        c = lax.axis_index("c")
        s = lax.axis_index("s")
        w = c * _NS + s
        base = w * _EW
        pltpu.sync_copy(src_hbm.at[pl.ds(base, _KF * _CH)], isrc_v)
        pltpu.sync_copy(src_hbm.at[pl.ds(base + _KF * _CH, _TL)], ist_v)
        pltpu.sync_copy(dst_hbm.at[pl.ds(base + _KF * _CH, _TL)], idt_v)
        sl = pl.ds(s * _RPS, _RPS)
        pltpu.sync_copy(z_hbm.at[sl], acc_sh.at[sl])
        plsc.subcore_barrier()

        def gsrc(j):
            return feat_hbm.at[isrc_v.at[pl.ds(j * _CH, _CH)]]

        pltpu.async_copy(dst_hbm.at[pl.ds(base, _CH)], id0_v, dsem0)
        pltpu.async_copy(dst_hbm.at[pl.ds(base + _CH, _CH)], id1_v, dsem1)
        pltpu.async_copy(gsrc(0), rows0_v, gsem0)
        pltpu.async_copy(gsrc(1), rows1_v, gsem1)

        def half(j, id_v, rows_v, gsem, dsem):
            pltpu.make_async_copy(
                dst_hbm.at[pl.ds(base + j * _CH, _CH)], id_v, dsem).wait()
            pltpu.make_async_copy(gsrc(j), rows_v, gsem).wait()
            pltpu.sync_copy(rows_v, acc_sh.at[id_v], add=True)

            @pl.when(j + 2 < _KF)
            def _():
                pltpu.async_copy(
                    dst_hbm.at[pl.ds(base + (j + 2) * _CH, _CH)], id_v, dsem)
                pltpu.async_copy(gsrc(j + 2), rows_v, gsem)

        def step(jj, carry):
            half(2 * jj, id0_v, rows0_v, gsem0, dsem0)
            half(2 * jj + 1, id1_v, rows1_v, gsem1, dsem1)
            return carry

        lax.fori_loop(0, _KF // 2, step, 0)
        pltpu.async_copy(feat_hbm.at[ist_v], rowst_v, gsem0).wait()
        pltpu.sync_copy(rowst_v, acc_sh.at[idt_v], add=True)
        plsc.subcore_barrier()
        pltpu.sync_copy(acc_sh.at[sl], out_hbm.at[c, sl])

    return k(src, dst, feat, zeros)


def _mm_tc(xp, W1):
    def body(x_ref, w_ref, h_ref):
        h_ref[...] = jnp.dot(x_ref[...], w_ref[...],
                             preferred_element_type=jnp.float32)

    return pl.pallas_call(
        body,
        grid=(_NP // _RB,),
        in_specs=[
            pl.BlockSpec((_RB, _D), lambda i: (i, 0)),
            pl.BlockSpec((_D, _D), lambda i: (0, 0)),
        ],
        out_specs=pl.BlockSpec((_RB, _D), lambda i: (i, 0)),
        out_shape=jax.ShapeDtypeStruct((_NP, _D), jnp.float32),
    )(xp, W1)


def _scale_tc(h, degT):
    def body(h_ref, deg_ref, hs_ref):
        dsum = deg_ref[:, 0:1] + deg_ref[:, 1:2]
        dinv = lax.rsqrt(dsum + 1.0)
        hs_ref[...] = h_ref[...] * dinv

    return pl.pallas_call(
        body,
        grid=(_NP // _RB,),
        in_specs=[
            pl.BlockSpec((_RB, _D), lambda i: (i, 0)),
            pl.BlockSpec((_RB, _NC), lambda i: (i, 0)),
        ],
        out_specs=pl.BlockSpec((_RB, _D), lambda i: (i, 0)),
        out_shape=jax.ShapeDtypeStruct((_NP, _D), jnp.float32),
    )(h, degT)


def _layer2_tc(p0, p1, hs, degT, W2p, b1r):
    def body(p0_ref, p1_ref, hs_ref, deg_ref, w2_ref, b1_ref, qs_ref):
        dsum = deg_ref[:, 0:1] + deg_ref[:, 1:2]
        dinv = lax.rsqrt(dsum + 1.0)
        pre = (p0_ref[...] + p1_ref[...] + hs_ref[...]) * dinv + b1_ref[...]
        xb1 = jnp.maximum(pre, 0.0)
        q = jnp.dot(xb1, w2_ref[...], preferred_element_type=jnp.float32)
        qs_ref[...] = q * dinv

    return pl.pallas_call(
        body,
        grid=(_NP // _RB,),
        in_specs=[
            pl.BlockSpec((_RB, _D), lambda i: (i, 0)),
            pl.BlockSpec((_RB, _D), lambda i: (i, 0)),
            pl.BlockSpec((_RB, _D), lambda i: (i, 0)),
            pl.BlockSpec((_RB, _NC), lambda i: (i, 0)),
            pl.BlockSpec((_D, _OW), lambda i: (0, 0)),
            pl.BlockSpec((1, _D), lambda i: (0, 0)),
        ],
        out_specs=pl.BlockSpec((_RB, _OW), lambda i: (i, 0)),
        out_shape=jax.ShapeDtypeStruct((_NP, _OW), jnp.float32),
    )(p0, p1, hs, degT, W2p, b1r)


def _final_tc(q0, q1, qs, degT, b2r):
    fb = 400

    def body(q0_ref, q1_ref, qs_ref, deg_ref, b2_ref, out_ref):
        dsum = deg_ref[:, 0:1] + deg_ref[:, 1:2]
        dinv = lax.rsqrt(dsum + 1.0)
        r = (q0_ref[...] + q1_ref[...] + qs_ref[...]) * dinv + b2_ref[...]
        out_ref[...] = r[:, :_OD]

    return pl.pallas_call(
        body,
        grid=(_N // fb,),
        in_specs=[
            pl.BlockSpec((fb, _OW), lambda i: (i, 0)),
            pl.BlockSpec((fb, _OW), lambda i: (i, 0)),
            pl.BlockSpec((fb, _OW), lambda i: (i, 0)),
            pl.BlockSpec((fb, _NC), lambda i: (i, 0)),
            pl.BlockSpec((1, _OW), lambda i: (0, 0)),
        ],
        out_specs=pl.BlockSpec((fb, _OD), lambda i: (i, 0)),
        out_shape=jax.ShapeDtypeStruct((_N, _OD), jnp.float32),
    )(q0, q1, qs, degT, b2r)


def kernel(x, edge_index, W1, b1, W2, b2):
    ei = edge_index.astype(jnp.int32)
    xp = jnp.zeros((_NP, _D), jnp.float32).at[:_N].set(x)
    z1 = jnp.zeros((_NP,), jnp.float32)
    zD = jnp.zeros((_NP, _D), jnp.float32)
    zO = jnp.zeros((_NP, _OW), jnp.float32)
    one = jnp.ones((_CH,), jnp.float32)
    W2p = jnp.zeros((_D, _OW), jnp.float32).at[:, :_OD].set(W2)
    b1r = b1.reshape(1, _D)
    b2r = jnp.zeros((1, _OW), jnp.float32).at[0, :_OD].set(b2)

    src1, dst1 = ei[0], ei[1]
    degs = _deg_sc(dst1, z1, one)
    degT = degs.T
    h = _mm_tc(xp, W1)
    hs = _scale_tc(h, degT)
    parts = _scatter_sc(src1, dst1, hs, zD, _D)
    qs = _layer2_tc(parts[0], parts[1], hs, degT, W2p, b1r)
    parts2 = _scatter_sc(src1, dst1, qs, zO, _OW)
    return _final_tc(parts2[0], parts2[1], qs, degT, b2r)


# speedup vs baseline: 1.0422x; 1.0422x over previous
import functools

import jax
import jax.numpy as jnp
from jax import lax
from jax.experimental import pallas as pl
from jax.experimental.pallas import tpu as pltpu
from jax.experimental.pallas import tpu_sc as plsc

_N = 10000
_D = 128
_OD = 2
_NP = 10240
_NC, _NS = 2, 16
_NW = _NC * _NS
_CH = 128
_E = 320000
_NCH = _E // _CH
_KW = _NCH // _NW
_XW = _NCH - _KW * _NW
_OW = 16
_RB = 512
_RPS = _NP // _NS


def _mesh():
    return plsc.VectorSubcoreMesh(core_axis_name="c", subcore_axis_name="s",
                                  num_cores=_NC, num_subcores=_NS)


def _deg_sc(ei, z1, one):
    @functools.partial(
        pl.kernel,
        out_type=jax.ShapeDtypeStruct((_NC, _NP), jnp.float32),
        mesh=_mesh(),
        scratch_types=[
            pltpu.VMEM((_CH,), jnp.float32),
            pltpu.VMEM((2, _CH), jnp.int32),
            pltpu.VMEM((2, _CH), jnp.int32),
            pltpu.SemaphoreType.DMA,
            pltpu.SemaphoreType.DMA,
            pltpu.VMEM_SHARED((_NP,), jnp.float32),
        ],
    )
    def k(ei_hbm, z_hbm, one_hbm, out_hbm,
          ones_v, ib0, ib1, dsem0, dsem1, deg_sh):
        c = lax.axis_index("c")
        s = lax.axis_index("s")
        w = c * _NS + s

        def echunk(j):
            off = pl.multiple_of((w * _KW + j) * _CH, _CH)
            return ei_hbm.at[:, pl.ds(off, _CH)]

        pltpu.sync_copy(one_hbm, ones_v)
        sl = pl.ds(s * _RPS, _RPS)
        pltpu.sync_copy(z_hbm.at[sl], deg_sh.at[sl])
        plsc.subcore_barrier()

        pltpu.async_copy(echunk(0), ib0, dsem0)
        pltpu.async_copy(echunk(1), ib1, dsem1)

        def half(j, ib, dsem):
            pltpu.make_async_copy(echunk(j), ib, dsem).wait()
            pltpu.sync_copy(ones_v, deg_sh.at[ib.at[1]], add=True)

            @pl.when(j + 2 < _KW)
            def _():
                pltpu.async_copy(echunk(j + 2), ib, dsem)

        def step(jj, carry):
            half(2 * jj, ib0, dsem0)
            half(2 * jj + 1, ib1, dsem1)
            return carry

        lax.fori_loop(0, _KW // 2, step, 0)

        @pl.when(w < _XW)
        def _():
            off = pl.multiple_of((_NW * _KW + w) * _CH, _CH)
            pltpu.sync_copy(ei_hbm.at[:, pl.ds(off, _CH)], ib0)
            pltpu.sync_copy(ones_v, deg_sh.at[ib0.at[1]], add=True)

        plsc.subcore_barrier()
        pltpu.sync_copy(deg_sh.at[sl], out_hbm.at[c, sl])

    return k(ei, z1, one)


def _scatter_sc(ei, feat, zeros, width):
    @functools.partial(
        pl.kernel,
        out_type=jax.ShapeDtypeStruct((_NC, _NP, width), jnp.float32),
        mesh=_mesh(),
        compiler_params=pltpu.CompilerParams(
            use_tc_tiling_on_sc=(width == _D)),
        scratch_types=[
            pltpu.VMEM((2, _CH), jnp.int32),
            pltpu.VMEM((2, _CH), jnp.int32),
            pltpu.VMEM((2, _CH), jnp.int32),
            pltpu.VMEM((2, _CH), jnp.int32),
            pltpu.VMEM((_CH, width), jnp.float32),
            pltpu.VMEM((_CH, width), jnp.float32),
            pltpu.SemaphoreType.DMA,
            pltpu.SemaphoreType.DMA,
            pltpu.SemaphoreType.DMA,
            pltpu.SemaphoreType.DMA,
            pltpu.SemaphoreType.DMA,
            pltpu.SemaphoreType.DMA,
            pltpu.VMEM_SHARED((_NP, width), jnp.float32),
        ],
    )
    def k(ei_hbm, feat_hbm, z_hbm, out_hbm,
          ib00, ib01, ib10, ib11, rows0_v, rows1_v,
          is00, is01, is10, is11, gsem0, gsem1, acc_sh):
        c = lax.axis_index("c")
        s = lax.axis_index("s")
        w = c * _NS + s

        def echunk(j):
            off = pl.multiple_of((w * _KW + j) * _CH, _CH)
            return ei_hbm.at[:, pl.ds(off, _CH)]

        sl = pl.ds(s * _RPS, _RPS)
        pltpu.sync_copy(z_hbm.at[sl], acc_sh.at[sl])
        plsc.subcore_barrier()

        ibs = ((ib00, ib01), (ib10, ib11))
        iss = ((is00, is01), (is10, is11))
        rows = (rows0_v, rows1_v)
        gsems = (gsem0, gsem1)

        pltpu.async_copy(echunk(0), ib00, is00)
        pltpu.async_copy(echunk(1), ib10, is10)
        pltpu.async_copy(echunk(2), ib01, is01)
        pltpu.async_copy(echunk(3), ib11, is11)
        pltpu.make_async_copy(echunk(0), ib00, is00).wait()
        pltpu.async_copy(feat_hbm.at[ib00.at[0]], rows0_v, gsem0)
        pltpu.make_async_copy(echunk(1), ib10, is10).wait()
        pltpu.async_copy(feat_hbm.at[ib10.at[0]], rows1_v, gsem1)

        def half(jj, p, r):
            j = 2 * jj + p
            ib_a, sem_a = ibs[p][r], iss[p][r]
            ib_b, sem_b = ibs[p][1 - r], iss[p][1 - r]
            pltpu.make_async_copy(feat_hbm.at[ib_a.at[0]], rows[p],
                                  gsems[p]).wait()
            pltpu.sync_copy(rows[p], acc_sh.at[ib_a.at[1]], add=True)

            @pl.when(jj < _KW // 2 - 2)
            def _():
                pltpu.async_copy(echunk(j + 4), ib_a, sem_a)

            @pl.when(jj < _KW // 2 - 1)
            def _():
                pltpu.make_async_copy(echunk(j + 2), ib_b, sem_b).wait()
                pltpu.async_copy(feat_hbm.at[ib_b.at[0]], rows[p], gsems[p])

        def step(m, carry):
            jj0 = 2 * m
            half(jj0, 0, 0)
            half(jj0, 1, 0)
            half(jj0 + 1, 0, 1)
            half(jj0 + 1, 1, 1)
            return carry

        lax.fori_loop(0, _KW // 4, step, 0)
        for jj in range(2 * (_KW // 4), _KW // 2):
            half(jj, 0, jj % 2)
            half(jj, 1, jj % 2)

        @pl.when(w < _XW)
        def _():
            off = pl.multiple_of((_NW * _KW + w) * _CH, _CH)
            pltpu.sync_copy(ei_hbm.at[:, pl.ds(off, _CH)], ib00)
            pltpu.async_copy(feat_hbm.at[ib00.at[0]], rows0_v, gsem0).wait()
            pltpu.sync_copy(rows0_v, acc_sh.at[ib00.at[1]], add=True)

        plsc.subcore_barrier()
        pltpu.sync_copy(acc_sh.at[sl], out_hbm.at[c, sl])

    return k(ei, feat, zeros)


def _mm_tc(xp, W1):
    def body(x_ref, w_ref, h_ref):
        h_ref[...] = jnp.dot(x_ref[...], w_ref[...],
                             preferred_element_type=jnp.float32)

    return pl.pallas_call(
        body,
        grid=(_NP // _RB,),
        in_specs=[
            pl.BlockSpec((_RB, _D), lambda i: (i, 0)),
            pl.BlockSpec((_D, _D), lambda i: (0, 0)),
        ],
        out_specs=pl.BlockSpec((_RB, _D), lambda i: (i, 0)),
        out_shape=jax.ShapeDtypeStruct((_NP, _D), jnp.float32),
    )(xp, W1)


def _scale_tc(h, degT):
    def body(h_ref, deg_ref, hs_ref):
        dsum = deg_ref[:, 0:1] + deg_ref[:, 1:2]
        dinv = lax.rsqrt(dsum + 1.0)
        hs_ref[...] = h_ref[...] * dinv

    return pl.pallas_call(
        body,
        grid=(_NP // _RB,),
        in_specs=[
            pl.BlockSpec((_RB, _D), lambda i: (i, 0)),
            pl.BlockSpec((_RB, _NC), lambda i: (i, 0)),
        ],
        out_specs=pl.BlockSpec((_RB, _D), lambda i: (i, 0)),
        out_shape=jax.ShapeDtypeStruct((_NP, _D), jnp.float32),
    )(h, degT)


def _layer2_tc(p0, p1, hs, degT, W2p, b1r):
    def body(p0_ref, p1_ref, hs_ref, deg_ref, w2_ref, b1_ref, qs_ref):
        dsum = deg_ref[:, 0:1] + deg_ref[:, 1:2]
        dinv = lax.rsqrt(dsum + 1.0)
        pre = (p0_ref[...] + p1_ref[...] + hs_ref[...]) * dinv + b1_ref[...]
        xb1 = jnp.maximum(pre, 0.0)
        q = jnp.dot(xb1, w2_ref[...], preferred_element_type=jnp.float32)
        qs_ref[...] = q * dinv

    return pl.pallas_call(
        body,
        grid=(_NP // _RB,),
        in_specs=[
            pl.BlockSpec((_RB, _D), lambda i: (i, 0)),
            pl.BlockSpec((_RB, _D), lambda i: (i, 0)),
            pl.BlockSpec((_RB, _D), lambda i: (i, 0)),
            pl.BlockSpec((_RB, _NC), lambda i: (i, 0)),
            pl.BlockSpec((_D, _OW), lambda i: (0, 0)),
            pl.BlockSpec((1, _D), lambda i: (0, 0)),
        ],
        out_specs=pl.BlockSpec((_RB, _OW), lambda i: (i, 0)),
        out_shape=jax.ShapeDtypeStruct((_NP, _OW), jnp.float32),
    )(p0, p1, hs, degT, W2p, b1r)


def _final_tc(q0, q1, qs, degT, b2r):
    fb = 400

    def body(q0_ref, q1_ref, qs_ref, deg_ref, b2_ref, out_ref):
        dsum = deg_ref[:, 0:1] + deg_ref[:, 1:2]
        dinv = lax.rsqrt(dsum + 1.0)
        r = (q0_ref[...] + q1_ref[...] + qs_ref[...]) * dinv + b2_ref[...]
        out_ref[...] = r[:, :_OD]

    return pl.pallas_call(
        body,
        grid=(_N // fb,),
        in_specs=[
            pl.BlockSpec((fb, _OW), lambda i: (i, 0)),
            pl.BlockSpec((fb, _OW), lambda i: (i, 0)),
            pl.BlockSpec((fb, _OW), lambda i: (i, 0)),
            pl.BlockSpec((fb, _NC), lambda i: (i, 0)),
            pl.BlockSpec((1, _OW), lambda i: (0, 0)),
        ],
        out_specs=pl.BlockSpec((fb, _OD), lambda i: (i, 0)),
        out_shape=jax.ShapeDtypeStruct((_N, _OD), jnp.float32),
    )(q0, q1, qs, degT, b2r)


def kernel(x, edge_index, W1, b1, W2, b2):
    ei = edge_index.astype(jnp.int32)
    xp = jnp.zeros((_NP, _D), jnp.float32).at[:_N].set(x)
    z1 = jnp.zeros((_NP,), jnp.float32)
    zD = jnp.zeros((_NP, _D), jnp.float32)
    zO = jnp.zeros((_NP, _OW), jnp.float32)
    one = jnp.ones((_CH,), jnp.float32)
    W2p = jnp.zeros((_D, _OW), jnp.float32).at[:, :_OD].set(W2)
    b1r = b1.reshape(1, _D)
    b2r = jnp.zeros((1, _OW), jnp.float32).at[0, :_OD].set(b2)

    degs = _deg_sc(ei, z1, one)
    degT = degs.T
    h = _mm_tc(xp, W1)
    hs = _scale_tc(h, degT)
    parts = _scatter_sc(ei, hs, zD, _D)
    qs = _layer2_tc(parts[0], parts[1], hs, degT, W2p, b1r)
    parts2 = _scatter_sc(ei, qs, zO, _OW)
    return _final_tc(parts2[0], parts2[1], qs, degT, b2r)


# speedup vs baseline: 1.1508x; 1.1041x over previous
import functools

import jax
import jax.numpy as jnp
from jax import lax
from jax.experimental import pallas as pl
from jax.experimental.pallas import tpu as pltpu
from jax.experimental.pallas import tpu_sc as plsc

_N = 10000
_D = 128
_OD = 2
_NP = 10240
_NC, _NS = 2, 16
_NW = _NC * _NS
_CH = 128
_E = 320000
_NCH = _E // _CH
_KW = _NCH // _NW
_XW = _NCH - _KW * _NW
_OW = 16
_RB = 512
_RPS = _NP // _NS


def _mesh():
    return plsc.VectorSubcoreMesh(core_axis_name="c", subcore_axis_name="s",
                                  num_cores=_NC, num_subcores=_NS)


def _deg_sc(ei, z1, one):
    @functools.partial(
        pl.kernel,
        out_type=jax.ShapeDtypeStruct((_NC, _NP), jnp.float32),
        mesh=_mesh(),
        scratch_types=[
            pltpu.VMEM((_CH,), jnp.float32),
            pltpu.VMEM((_KW, 2, _CH), jnp.int32),
            pltpu.SemaphoreType.DMA,
            pltpu.VMEM_SHARED((_NP,), jnp.float32),
        ],
    )
    def k(ei_hbm, z_hbm, one_hbm, out_hbm,
          ones_v, idx_v, dsem, deg_sh):
        c = lax.axis_index("c")
        s = lax.axis_index("s")
        w = c * _NS + s

        def echunk(j):
            off = pl.multiple_of((w * _KW + j) * _CH, _CH)
            return ei_hbm.at[:, pl.ds(off, _CH)]

        pltpu.sync_copy(one_hbm, ones_v)
        sl = pl.ds(s * _RPS, _RPS)
        pltpu.sync_copy(z_hbm.at[sl], deg_sh.at[sl])

        def fire(j, carry):
            pltpu.async_copy(echunk(j), idx_v.at[j], dsem)
            return carry

        lax.fori_loop(0, _KW, fire, 0)
        plsc.subcore_barrier()

        def drain(j, carry):
            pltpu.make_async_copy(echunk(j), idx_v.at[j], dsem).wait()
            return carry

        lax.fori_loop(0, _KW, drain, 0)

        def scat(j, carry):
            pltpu.sync_copy(ones_v, deg_sh.at[idx_v.at[j, 1]], add=True)
            return carry

        lax.fori_loop(0, _KW, scat, 0)

        @pl.when(w < _XW)
        def _():
            off = pl.multiple_of((_NW * _KW + w) * _CH, _CH)
            pltpu.sync_copy(ei_hbm.at[:, pl.ds(off, _CH)], idx_v.at[0])
            pltpu.sync_copy(ones_v, deg_sh.at[idx_v.at[0, 1]], add=True)

        plsc.subcore_barrier()
        pltpu.sync_copy(deg_sh.at[sl], out_hbm.at[c, sl])

    return k(ei, z1, one)


def _scatter_sc(ei, feat, zeros, width):
    @functools.partial(
        pl.kernel,
        out_type=(jax.ShapeDtypeStruct((_NP, width), jnp.float32),
                  jax.ShapeDtypeStruct((_NP, width), jnp.float32)),
        mesh=_mesh(),
        compiler_params=pltpu.CompilerParams(
            use_tc_tiling_on_sc=(width == _D)),
        scratch_types=[
            pltpu.VMEM((2, _CH), jnp.int32),
            pltpu.VMEM((2, _CH), jnp.int32),
            pltpu.VMEM((2, _CH), jnp.int32),
            pltpu.VMEM((2, _CH), jnp.int32),
            pltpu.VMEM((_CH, width), jnp.float32),
            pltpu.VMEM((_CH, width), jnp.float32),
            pltpu.SemaphoreType.DMA,
            pltpu.SemaphoreType.DMA,
            pltpu.SemaphoreType.DMA,
            pltpu.SemaphoreType.DMA,
            pltpu.SemaphoreType.DMA,
            pltpu.SemaphoreType.DMA,
            pltpu.VMEM_SHARED((_NP, width), jnp.float32),
        ],
    )
    def k(ei_hbm, feat_hbm, z_hbm, out0_hbm, out1_hbm,
          ib00, ib01, ib10, ib11, rows0_v, rows1_v,
          is00, is01, is10, is11, gsem0, gsem1, acc_sh):
        c = lax.axis_index("c")
        s = lax.axis_index("s")
        w = c * _NS + s

        def echunk(j):
            off = pl.multiple_of((w * _KW + j) * _CH, _CH)
            return ei_hbm.at[:, pl.ds(off, _CH)]

        sl = pl.ds(s * _RPS, _RPS)
        pltpu.sync_copy(z_hbm.at[sl], acc_sh.at[sl])
        plsc.subcore_barrier()

        ibs = ((ib00, ib01), (ib10, ib11))
        iss = ((is00, is01), (is10, is11))
        rows = (rows0_v, rows1_v)
        gsems = (gsem0, gsem1)

        pltpu.async_copy(echunk(0), ib00, is00)
        pltpu.async_copy(echunk(1), ib10, is10)
        pltpu.async_copy(echunk(2), ib01, is01)
        pltpu.async_copy(echunk(3), ib11, is11)
        pltpu.make_async_copy(echunk(0), ib00, is00).wait()
        pltpu.async_copy(feat_hbm.at[ib00.at[0]], rows0_v, gsem0)
        pltpu.make_async_copy(echunk(1), ib10, is10).wait()
        pltpu.async_copy(feat_hbm.at[ib10.at[0]], rows1_v, gsem1)

        def half(jj, p, r):
            j = 2 * jj + p
            ib_a, sem_a = ibs[p][r], iss[p][r]
            ib_b, sem_b = ibs[p][1 - r], iss[p][1 - r]
            pltpu.make_async_copy(feat_hbm.at[ib_a.at[0]], rows[p],
                                  gsems[p]).wait()
            pltpu.sync_copy(rows[p], acc_sh.at[ib_a.at[1]], add=True)

            @pl.when(jj < _KW // 2 - 2)
            def _():
                pltpu.async_copy(echunk(j + 4), ib_a, sem_a)

            @pl.when(jj < _KW // 2 - 1)
            def _():
                pltpu.make_async_copy(echunk(j + 2), ib_b, sem_b).wait()
                pltpu.async_copy(feat_hbm.at[ib_b.at[0]], rows[p], gsems[p])

        def step(m, carry):
            jj0 = 2 * m
            half(jj0, 0, 0)
            half(jj0, 1, 0)
            half(jj0 + 1, 0, 1)
            half(jj0 + 1, 1, 1)
            return carry

        lax.fori_loop(0, _KW // 4, step, 0)
        for jj in range(2 * (_KW // 4), _KW // 2):
            half(jj, 0, jj % 2)
            half(jj, 1, jj % 2)

        @pl.when(w < _XW)
        def _():
            off = pl.multiple_of((_NW * _KW + w) * _CH, _CH)
            pltpu.sync_copy(ei_hbm.at[:, pl.ds(off, _CH)], ib00)
            pltpu.async_copy(feat_hbm.at[ib00.at[0]], rows0_v, gsem0).wait()
            pltpu.sync_copy(rows0_v, acc_sh.at[ib00.at[1]], add=True)

        plsc.subcore_barrier()

        @pl.when(c == 0)
        def _():
            pltpu.sync_copy(acc_sh.at[sl], out0_hbm.at[sl])

        @pl.when(c == 1)
        def _():
            pltpu.sync_copy(acc_sh.at[sl], out1_hbm.at[sl])

    return k(ei, feat, zeros)


def _mm_tc(xp, W1):
    def body(x_ref, w_ref, h_ref):
        h_ref[...] = jnp.dot(x_ref[...], w_ref[...],
                             preferred_element_type=jnp.float32)

    return pl.pallas_call(
        body,
        grid=(_NP // _RB,),
        in_specs=[
            pl.BlockSpec((_RB, _D), lambda i: (i, 0)),
            pl.BlockSpec((_D, _D), lambda i: (0, 0)),
        ],
        out_specs=pl.BlockSpec((_RB, _D), lambda i: (i, 0)),
        out_shape=jax.ShapeDtypeStruct((_NP, _D), jnp.float32),
    )(xp, W1)


def _scale_tc(h, degT):
    def body(h_ref, deg_ref, hs_ref):
        dsum = deg_ref[:, 0:1] + deg_ref[:, 1:2]
        dinv = lax.rsqrt(dsum + 1.0)
        hs_ref[...] = h_ref[...] * dinv

    return pl.pallas_call(
        body,
        grid=(_NP // _RB,),
        in_specs=[
            pl.BlockSpec((_RB, _D), lambda i: (i, 0)),
            pl.BlockSpec((_RB, _NC), lambda i: (i, 0)),
        ],
        out_specs=pl.BlockSpec((_RB, _D), lambda i: (i, 0)),
        out_shape=jax.ShapeDtypeStruct((_NP, _D), jnp.float32),
    )(h, degT)


def _layer2_tc(p0, p1, hs, degT, W2p, b1r):
    def body(p0_ref, p1_ref, hs_ref, deg_ref, w2_ref, b1_ref, qs_ref):
        dsum = deg_ref[:, 0:1] + deg_ref[:, 1:2]
        dinv = lax.rsqrt(dsum + 1.0)
        pre = (p0_ref[...] + p1_ref[...] + hs_ref[...]) * dinv + b1_ref[...]
        xb1 = jnp.maximum(pre, 0.0)
        q = jnp.dot(xb1, w2_ref[...], preferred_element_type=jnp.float32)
        qs_ref[...] = q * dinv

    return pl.pallas_call(
        body,
        grid=(_NP // _RB,),
        in_specs=[
            pl.BlockSpec((_RB, _D), lambda i: (i, 0)),
            pl.BlockSpec((_RB, _D), lambda i: (i, 0)),
            pl.BlockSpec((_RB, _D), lambda i: (i, 0)),
            pl.BlockSpec((_RB, _NC), lambda i: (i, 0)),
            pl.BlockSpec((_D, _OW), lambda i: (0, 0)),
            pl.BlockSpec((1, _D), lambda i: (0, 0)),
        ],
        out_specs=pl.BlockSpec((_RB, _OW), lambda i: (i, 0)),
        out_shape=jax.ShapeDtypeStruct((_NP, _OW), jnp.float32),
    )(p0, p1, hs, degT, W2p, b1r)


def _final_tc(q0, q1, qs, degT, b2r):
    fb = 400

    def body(q0_ref, q1_ref, qs_ref, deg_ref, b2_ref, out_ref):
        dsum = deg_ref[:, 0:1] + deg_ref[:, 1:2]
        dinv = lax.rsqrt(dsum + 1.0)
        r = (q0_ref[...] + q1_ref[...] + qs_ref[...]) * dinv + b2_ref[...]
        out_ref[...] = r[:, :_OD]

    return pl.pallas_call(
        body,
        grid=(_N // fb,),
        in_specs=[
            pl.BlockSpec((fb, _OW), lambda i: (i, 0)),
            pl.BlockSpec((fb, _OW), lambda i: (i, 0)),
            pl.BlockSpec((fb, _OW), lambda i: (i, 0)),
            pl.BlockSpec((fb, _NC), lambda i: (i, 0)),
            pl.BlockSpec((1, _OW), lambda i: (0, 0)),
        ],
        out_specs=pl.BlockSpec((fb, _OD), lambda i: (i, 0)),
        out_shape=jax.ShapeDtypeStruct((_N, _OD), jnp.float32),
    )(q0, q1, qs, degT, b2r)


def kernel(x, edge_index, W1, b1, W2, b2):
    ei = edge_index.astype(jnp.int32)
    xp = jnp.zeros((_NP, _D), jnp.float32).at[:_N].set(x)
    z1 = jnp.zeros((_NP,), jnp.float32)
    zD = jnp.zeros((_NP, _D), jnp.float32)
    zO = jnp.zeros((_NP, _OW), jnp.float32)
    one = jnp.ones((_CH,), jnp.float32)
    W2p = jnp.zeros((_D, _OW), jnp.float32).at[:, :_OD].set(W2)
    b1r = b1.reshape(1, _D)
    b2r = jnp.zeros((1, _OW), jnp.float32).at[0, :_OD].set(b2)

    degs = _deg_sc(ei, z1, one)
    degT = degs.T
    h = _mm_tc(xp, W1)
    hs = _scale_tc(h, degT)
    p0, p1 = _scatter_sc(ei, hs, zD, _D)
    qs = _layer2_tc(p0, p1, hs, degT, W2p, b1r)
    q0, q1 = _scatter_sc(ei, qs, zO, _OW)
    return _final_tc(q0, q1, qs, degT, b2r)
